# baseline (speedup 1.0000x reference)
import functools

import jax
import jax.numpy as jnp
from jax import lax
from jax.experimental import pallas as pl
from jax.experimental.pallas import tpu as pltpu
from jax.experimental.pallas import tpu_sc as plsc

F = 26
V = 100000
D = 32
B = 4096
LF = 200
OR = F + LF

NC, NS, LANES = 2, 16, 16
NW = NC * NS
BPW = B // NW
PPW = BPW * F
CHUNK = 128
NCHUNK = PPW // CHUNK
VECS = PPW // LANES
VPC = CHUNK // LANES

_mesh = plsc.VectorSubcoreMesh(core_axis_name="c", subcore_axis_name="s")


@functools.partial(
    pl.kernel,
    out_type=jax.ShapeDtypeStruct((B * OR, D), jnp.float32),
    mesh=_mesh,
    compiler_params=pltpu.CompilerParams(use_tc_tiling_on_sc=False),
    scratch_types=[
        pltpu.VMEM((PPW,), jnp.int32),
        pltpu.VMEM((NCHUNK, CHUNK), jnp.int32),
        pltpu.VMEM((NCHUNK, CHUNK), jnp.int32),
        pltpu.VMEM((CHUNK, D), jnp.float32),
        pltpu.VMEM((LF + 8, D), jnp.float32),
        pltpu.SemaphoreType.DMA,
        pltpu.SemaphoreType.DMA,
    ],
)
def _sc_conditioning(tab_ref, idx_ref, feat_ref, out_ref,
                     idxin_v, src_v, dst_v, rows_v, fbuf_v, sem_g, sem_s):
    wid = lax.axis_index("s") * NC + lax.axis_index("c")
    p0 = pl.multiple_of(wid * PPW, 8)
    b0 = wid * BPW

    pltpu.sync_copy(idx_ref.at[pl.ds(p0, PPW)], idxin_v)

    def compute(i, carry):
        q = i * LANES + lax.iota(jnp.int32, LANES)
        b = lax.div(q, jnp.full((LANES,), F, jnp.int32))
        f = q - b * F
        raw = idxin_v[pl.ds(i * LANES, LANES)]
        j = i // VPC
        col = (i - j * VPC) * LANES
        src_v[j, pl.ds(col, LANES)] = raw + f * V
        dst_v[j, pl.ds(col, LANES)] = (b0 + b) * OR + f
        return carry

    lax.fori_loop(0, VECS, compute, 0)

    def feat_group(g, carry):
        for j in range(4):
            d = (2 * j + 2) % 8
            row = b0 + g * 4 + j
            src_off = pl.multiple_of(row * LF, 8)
            n = LF if d == 0 else LF + 8
            dst_off = pl.multiple_of(row * OR + F - d, 8)
            pltpu.sync_copy(feat_ref.at[pl.ds(src_off, LF)],
                            fbuf_v.at[pl.ds(d, LF)])
            pltpu.sync_copy(fbuf_v.at[pl.ds(0, n)],
                            out_ref.at[pl.ds(dst_off, n)])
        return carry

    lax.fori_loop(0, BPW // 4, feat_group, 0)

    def embed_chunk(c, carry):
        pltpu.async_copy(tab_ref.at[src_v.at[c]], rows_v, sem_g).wait()
        pltpu.async_copy(rows_v, out_ref.at[dst_v.at[c]], sem_s).wait()
        return carry

    lax.fori_loop(0, NCHUNK, embed_chunk, 0)


def kernel(feature, indices, tables):
    tab2 = tables.reshape(F * V, D)
    idx = indices.astype(jnp.int32).reshape(B * F)
    feat2 = feature.reshape(B * LF, D)
    out = _sc_conditioning(tab2, idx, feat2)
    return out.reshape(B, OR, D)
